# baseline (speedup 1.0000x reference)
import functools

import jax
import jax.numpy as jnp
from jax import lax
from jax.experimental import pallas as pl
from jax.experimental.pallas import tpu as pltpu
from jax.experimental.pallas import tpu_sc as plsc

N_NODES = 10000
D = 128
OUT_DIM = 32
NUM_LAYERS = 3
BN_EPS = 1e-5

NC = 2
NS = 16
NW = NC * NS

NPAD = 10240
NACC = 10112
ACC_ROWS_PER_TILE = NACC // NS

E_CHUNK = 128
N_EDGES = 320000
NBUF = 2
UNROLL = 10
EPW_CHUNKS = 80
NSTAGE = 2
STAGE_CHUNKS = EPW_CHUNKS // NSTAGE
SEGS_PER_STAGE = STAGE_CHUNKS // UNROLL
EPW = EPW_CHUNKS * E_CHUNK
EPAD = EPW * NW

_sc_mesh = plsc.VectorSubcoreMesh(core_axis_name="c", subcore_axis_name="s")


@functools.partial(
    pl.kernel,
    mesh=_sc_mesh,
    out_type=jax.ShapeDtypeStruct((NC, NACC, D), jnp.float32),
    scratch_types=[
        pltpu.VMEM((STAGE_CHUNKS, E_CHUNK), jnp.int32),
        pltpu.VMEM((STAGE_CHUNKS, E_CHUNK), jnp.int32),
        pltpu.VMEM((NBUF, E_CHUNK, D), jnp.float32),
        pltpu.VMEM_SHARED((NACC, D), jnp.float32),
        pltpu.SemaphoreType.DMA((NBUF,)),
    ],
)
def _sc_segment_sum(src_hbm, dst_hbm, h_hbm, zeros_hbm, out_hbm,
                    src_v, dst_v, rows_v, acc_sh, gsem):
    cid = lax.axis_index("c")
    sid = lax.axis_index("s")
    wid = sid * NC + cid
    chunk_base = wid * EPW_CHUNKS
    row_base = sid * ACC_ROWS_PER_TILE
    rows_sl = pl.ds(row_base, ACC_ROWS_PER_TILE)

    pltpu.sync_copy(zeros_hbm, acc_sh.at[rows_sl])
    plsc.subcore_barrier()

    for stage in range(NSTAGE):
        sbase = chunk_base + stage * STAGE_CHUNKS
        pltpu.sync_copy(src_hbm.at[pl.ds(sbase, STAGE_CHUNKS)], src_v)
        pltpu.sync_copy(dst_hbm.at[pl.ds(sbase, STAGE_CHUNKS)], dst_v)

        def segment(s, carry):
            j0 = s * UNROLL
            descs = [None] * UNROLL
            for i in range(NBUF):
                descs[i] = pltpu.async_copy(
                    h_hbm.at[src_v.at[j0 + i]], rows_v.at[i], gsem.at[i])
            for i in range(UNROLL):
                b = i % NBUF
                descs[i].wait()
                pltpu.sync_copy(rows_v.at[b], acc_sh.at[dst_v.at[j0 + i]],
                                add=True)
                if i + NBUF < UNROLL:
                    descs[i + NBUF] = pltpu.async_copy(
                        h_hbm.at[src_v.at[j0 + i + NBUF]], rows_v.at[b],
                        gsem.at[b])
            return carry

        lax.fori_loop(0, SEGS_PER_STAGE, segment, 0)

    plsc.subcore_barrier()
    pltpu.sync_copy(acc_sh.at[rows_sl], out_hbm.at[cid, rows_sl])


def _mlp_body(h_ref, p0_ref, p1_ref, w1_ref, b1_ref, w2_ref, b2_ref,
              sc_ref, sh_ref, out_ref):
    agg = h_ref[...] + p0_ref[...] + p1_ref[...]
    h1 = jnp.maximum(
        jnp.dot(agg, w1_ref[...], preferred_element_type=jnp.float32)
        + b1_ref[...], 0.0)
    h2 = (jnp.dot(h1, w2_ref[...], preferred_element_type=jnp.float32)
          + b2_ref[...])
    out_ref[...] = jnp.maximum(h2 * sc_ref[...] + sh_ref[...], 0.0)


def _mlp_final_body(h_ref, p0_ref, p1_ref, w1_ref, b1_ref, w2_ref, b2_ref,
                    sc_ref, sh_ref, wc_ref, bc_ref, out_ref, cls_ref):
    _mlp_body(h_ref, p0_ref, p1_ref, w1_ref, b1_ref, w2_ref, b2_ref,
              sc_ref, sh_ref, out_ref)
    cls_ref[...] = (jnp.dot(out_ref[...], wc_ref[...],
                            preferred_element_type=jnp.float32) + bc_ref[...])


_BLK = 1024
_row_spec = pl.BlockSpec((_BLK, D), lambda i: (i, 0))
_w_spec = pl.BlockSpec((D, D), lambda i: (0, 0))
_v_spec = pl.BlockSpec((1, D), lambda i: (0, 0))


def _tc_mlp(h, p0, p1, w1, b1, w2, b2, scale, shift):
    return pl.pallas_call(
        _mlp_body,
        grid=(NPAD // _BLK,),
        in_specs=[_row_spec, _row_spec, _row_spec, _w_spec, _v_spec,
                  _w_spec, _v_spec, _v_spec, _v_spec],
        out_specs=_row_spec,
        out_shape=jax.ShapeDtypeStruct((NPAD, D), jnp.float32),
    )(h, p0, p1, w1, b1, w2, b2, scale, shift)


def _tc_mlp_final(h, p0, p1, w1, b1, w2, b2, scale, shift, wc, bc):
    return pl.pallas_call(
        _mlp_final_body,
        grid=(NPAD // _BLK,),
        in_specs=[_row_spec, _row_spec, _row_spec, _w_spec, _v_spec,
                  _w_spec, _v_spec, _v_spec, _v_spec, _w_spec, _v_spec],
        out_specs=(_row_spec, _row_spec),
        out_shape=(jax.ShapeDtypeStruct((NPAD, D), jnp.float32),
                   jax.ShapeDtypeStruct((NPAD, D), jnp.float32)),
    )(h, p0, p1, w1, b1, w2, b2, scale, shift, wc, bc)


def _pad_parts(parts):
    return jnp.zeros((NC, NPAD, D), jnp.float32).at[:, :NACC].set(parts)


def kernel(x, edge_index, params):
    ei = edge_index.astype(jnp.int32)
    pad_e = EPAD - N_EDGES
    src = jnp.concatenate(
        [ei[0], jnp.full((pad_e,), N_NODES, dtype=jnp.int32)]
    ).reshape(EPAD // E_CHUNK, E_CHUNK)
    dst = jnp.concatenate(
        [ei[1], jnp.full((pad_e,), N_NODES, dtype=jnp.int32)]
    ).reshape(EPAD // E_CHUNK, E_CHUNK)

    h = jnp.zeros((NPAD, D), jnp.float32).at[:N_NODES].set(x)
    zeros = jnp.zeros((ACC_ROWS_PER_TILE, D), jnp.float32)

    for i in range(NUM_LAYERS):
        cp = params[f'conv{i}']
        bn = params[f'bn{i}']
        scale = (bn['gamma'] * lax.rsqrt(bn['var'] + BN_EPS)).reshape(1, D)
        shift = (bn['beta'] - bn['mean'] * scale[0]).reshape(1, D)
        b1 = cp['b1'].reshape(1, D)
        b2 = cp['b2'].reshape(1, D)

        parts = _pad_parts(_sc_segment_sum(src, dst, h, zeros))
        if i < NUM_LAYERS - 1:
            h = _tc_mlp(h, parts[0], parts[1], cp['W1'], b1,
                        cp['W2'], b2, scale, shift)
        else:
            wc = jnp.zeros((D, D), jnp.float32).at[:, :OUT_DIM].set(
                params['Wc'])
            bc = jnp.zeros((1, D), jnp.float32).at[0, :OUT_DIM].set(
                params['bc'])
            h, cls = _tc_mlp_final(h, parts[0], parts[1], cp['W1'], b1,
                                   cp['W2'], b2, scale, shift, wc, bc)
    return cls[:N_NODES, :OUT_DIM]


# speedup vs baseline: 1.6176x; 1.6176x over previous
import functools

import jax
import jax.numpy as jnp
from jax import lax
from jax.experimental import pallas as pl
from jax.experimental.pallas import tpu as pltpu
from jax.experimental.pallas import tpu_sc as plsc

N_NODES = 10000
D = 128
OUT_DIM = 32
NUM_LAYERS = 3
BN_EPS = 1e-5

NC = 2
NS = 16
NW = NC * NS

NPAD = 10112
ROWS_PER_TILE = NPAD // NS

E_CHUNK = 128
N_EDGES = 320000
EPW_CHUNKS = 80
NSTAGE = 2
STAGE_CHUNKS = EPW_CHUNKS // NSTAGE
EPW = EPW_CHUNKS * E_CHUNK
EPAD = EPW * NW

_sc_mesh = plsc.VectorSubcoreMesh(core_axis_name="c", subcore_axis_name="s")


@functools.partial(
    pl.kernel,
    mesh=_sc_mesh,
    out_type=(jax.ShapeDtypeStruct((NC, NPAD, D), jnp.float32),
              jax.ShapeDtypeStruct((EPAD, D), jnp.float32)),
    scratch_types=[
        pltpu.VMEM((STAGE_CHUNKS, E_CHUNK), jnp.int32),
        pltpu.VMEM((E_CHUNK, D), jnp.float32),
        pltpu.VMEM_SHARED((NPAD, D), jnp.float32),
        pltpu.SemaphoreType.DMA,
    ],
)
def _sc_segment_sum(src_hbm, dst_hbm, h_hbm, zeros_hbm, out_hbm, msg_hbm,
                    idx_v, rows_v, sp_buf, gsem):
    cid = lax.axis_index("c")
    sid = lax.axis_index("s")
    wid = sid * NC + cid
    chunk_base = wid * EPW_CHUNKS
    row_base = sid * ROWS_PER_TILE
    rows_sl = pl.ds(row_base, ROWS_PER_TILE)

    pltpu.sync_copy(h_hbm.at[rows_sl], sp_buf.at[rows_sl])
    plsc.subcore_barrier()

    for stage in range(NSTAGE):
        sbase = chunk_base + stage * STAGE_CHUNKS
        pltpu.sync_copy(src_hbm.at[pl.ds(sbase, STAGE_CHUNKS)], idx_v)

        def gather_chunk(j, carry):
            pltpu.async_copy(sp_buf.at[idx_v.at[j]], rows_v, gsem).wait()
            pltpu.sync_copy(
                rows_v, msg_hbm.at[pl.ds((sbase + j) * E_CHUNK, E_CHUNK)])
            return carry

        lax.fori_loop(0, STAGE_CHUNKS, gather_chunk, 0)

    plsc.subcore_barrier()
    pltpu.sync_copy(zeros_hbm, sp_buf.at[rows_sl])
    plsc.subcore_barrier()

    for stage in range(NSTAGE):
        sbase = chunk_base + stage * STAGE_CHUNKS
        pltpu.sync_copy(dst_hbm.at[pl.ds(sbase, STAGE_CHUNKS)], idx_v)

        def scatter_chunk(j, carry):
            pltpu.async_copy(
                msg_hbm.at[pl.ds((sbase + j) * E_CHUNK, E_CHUNK)], rows_v,
                gsem).wait()
            pltpu.sync_copy(rows_v, sp_buf.at[idx_v.at[j]], add=True)
            return carry

        lax.fori_loop(0, STAGE_CHUNKS, scatter_chunk, 0)

    plsc.subcore_barrier()
    pltpu.sync_copy(sp_buf.at[rows_sl], out_hbm.at[cid, rows_sl])


def _mlp_body(h_ref, p0_ref, p1_ref, w1_ref, b1_ref, w2_ref, b2_ref,
              sc_ref, sh_ref, out_ref):
    agg = h_ref[...] + p0_ref[...] + p1_ref[...]
    h1 = jnp.maximum(
        jnp.dot(agg, w1_ref[...], preferred_element_type=jnp.float32)
        + b1_ref[...], 0.0)
    h2 = (jnp.dot(h1, w2_ref[...], preferred_element_type=jnp.float32)
          + b2_ref[...])
    out_ref[...] = jnp.maximum(h2 * sc_ref[...] + sh_ref[...], 0.0)


def _mlp_final_body(h_ref, p0_ref, p1_ref, w1_ref, b1_ref, w2_ref, b2_ref,
                    sc_ref, sh_ref, wc_ref, bc_ref, out_ref, cls_ref):
    _mlp_body(h_ref, p0_ref, p1_ref, w1_ref, b1_ref, w2_ref, b2_ref,
              sc_ref, sh_ref, out_ref)
    cls_ref[...] = (jnp.dot(out_ref[...], wc_ref[...],
                            preferred_element_type=jnp.float32) + bc_ref[...])


_BLK = 1264
_row_spec = pl.BlockSpec((_BLK, D), lambda i: (i, 0))
_w_spec = pl.BlockSpec((D, D), lambda i: (0, 0))
_v_spec = pl.BlockSpec((1, D), lambda i: (0, 0))


def _tc_mlp(h, p0, p1, w1, b1, w2, b2, scale, shift):
    return pl.pallas_call(
        _mlp_body,
        grid=(NPAD // _BLK,),
        in_specs=[_row_spec, _row_spec, _row_spec, _w_spec, _v_spec,
                  _w_spec, _v_spec, _v_spec, _v_spec],
        out_specs=_row_spec,
        out_shape=jax.ShapeDtypeStruct((NPAD, D), jnp.float32),
    )(h, p0, p1, w1, b1, w2, b2, scale, shift)


def _tc_mlp_final(h, p0, p1, w1, b1, w2, b2, scale, shift, wc, bc):
    return pl.pallas_call(
        _mlp_final_body,
        grid=(NPAD // _BLK,),
        in_specs=[_row_spec, _row_spec, _row_spec, _w_spec, _v_spec,
                  _w_spec, _v_spec, _v_spec, _v_spec, _w_spec, _v_spec],
        out_specs=(_row_spec, _row_spec),
        out_shape=(jax.ShapeDtypeStruct((NPAD, D), jnp.float32),
                   jax.ShapeDtypeStruct((NPAD, D), jnp.float32)),
    )(h, p0, p1, w1, b1, w2, b2, scale, shift, wc, bc)


def kernel(x, edge_index, params):
    ei = edge_index.astype(jnp.int32)
    pad_e = EPAD - N_EDGES
    src = jnp.concatenate(
        [ei[0], jnp.full((pad_e,), N_NODES, dtype=jnp.int32)]
    ).reshape(EPAD // E_CHUNK, E_CHUNK)
    dst = jnp.concatenate(
        [ei[1], jnp.full((pad_e,), N_NODES, dtype=jnp.int32)]
    ).reshape(EPAD // E_CHUNK, E_CHUNK)

    h = jnp.zeros((NPAD, D), jnp.float32).at[:N_NODES].set(x)
    zeros = jnp.zeros((ROWS_PER_TILE, D), jnp.float32)

    for i in range(NUM_LAYERS):
        cp = params[f'conv{i}']
        bn = params[f'bn{i}']
        scale = (bn['gamma'] * lax.rsqrt(bn['var'] + BN_EPS)).reshape(1, D)
        shift = (bn['beta'] - bn['mean'] * scale[0]).reshape(1, D)
        b1 = cp['b1'].reshape(1, D)
        b2 = cp['b2'].reshape(1, D)

        parts, _ = _sc_segment_sum(src, dst, h, zeros)
        if i < NUM_LAYERS - 1:
            h = _tc_mlp(h, parts[0], parts[1], cp['W1'], b1,
                        cp['W2'], b2, scale, shift)
        else:
            wc = jnp.zeros((D, D), jnp.float32).at[:, :OUT_DIM].set(
                params['Wc'])
            bc = jnp.zeros((1, D), jnp.float32).at[0, :OUT_DIM].set(
                params['bc'])
            h, cls = _tc_mlp_final(h, parts[0], parts[1], cp['W1'], b1,
                                   cp['W2'], b2, scale, shift, wc, bc)
    return cls[:N_NODES, :OUT_DIM]


# speedup vs baseline: 1.7841x; 1.1030x over previous
import functools

import jax
import jax.numpy as jnp
from jax import lax
from jax.experimental import pallas as pl
from jax.experimental.pallas import tpu as pltpu
from jax.experimental.pallas import tpu_sc as plsc

N_NODES = 10000
D = 128
OUT_DIM = 32
NUM_LAYERS = 3
BN_EPS = 1e-5

NC = 2
NS = 16
NW = NC * NS

NPAD = 10112
ROWS_PER_TILE = NPAD // NS

E_CHUNK = 128
N_EDGES = 320000
EPW_CHUNKS = 80
NSTAGE = 2
STAGE_CHUNKS = EPW_CHUNKS // NSTAGE
EPW = EPW_CHUNKS * E_CHUNK
EPAD = EPW * NW

_sc_mesh = plsc.VectorSubcoreMesh(core_axis_name="c", subcore_axis_name="s")


@functools.partial(
    pl.kernel,
    mesh=_sc_mesh,
    out_type=(jax.ShapeDtypeStruct((NC, NPAD, D), jnp.float32),
              jax.ShapeDtypeStruct((EPAD, D), jnp.float32)),
    scratch_types=[
        pltpu.VMEM((STAGE_CHUNKS, E_CHUNK), jnp.int32),
        pltpu.VMEM((2 * E_CHUNK, D), jnp.float32),
        pltpu.VMEM_SHARED((NPAD, D), jnp.float32),
        pltpu.SemaphoreType.DMA,
        pltpu.SemaphoreType.DMA,
    ],
)
def _sc_segment_sum(src_hbm, dst_hbm, h_hbm, zeros_hbm, out_hbm, msg_hbm,
                    idx_v, rows_v, sp_buf, gsem, gsem2):
    cid = lax.axis_index("c")
    sid = lax.axis_index("s")
    wid = sid * NC + cid
    chunk_base = wid * EPW_CHUNKS
    row_base = sid * ROWS_PER_TILE
    rows_sl = pl.ds(row_base, ROWS_PER_TILE)

    pltpu.sync_copy(h_hbm.at[rows_sl], sp_buf.at[rows_sl])
    plsc.subcore_barrier()

    for stage in range(NSTAGE):
        sbase = chunk_base + stage * STAGE_CHUNKS
        pltpu.sync_copy(src_hbm.at[pl.ds(sbase, STAGE_CHUNKS)], idx_v)

        def gather_pair(g, carry):
            d0 = pltpu.async_copy(sp_buf.at[idx_v.at[2 * g]],
                                  rows_v.at[pl.ds(0, E_CHUNK)], gsem)
            d1 = pltpu.async_copy(sp_buf.at[idx_v.at[2 * g + 1]],
                                  rows_v.at[pl.ds(E_CHUNK, E_CHUNK)], gsem2)
            d0.wait()
            d1.wait()
            pltpu.sync_copy(
                rows_v,
                msg_hbm.at[pl.ds((sbase + 2 * g) * E_CHUNK, 2 * E_CHUNK)])
            return carry

        lax.fori_loop(0, STAGE_CHUNKS // 2, gather_pair, 0)

    plsc.subcore_barrier()
    pltpu.sync_copy(zeros_hbm, sp_buf.at[rows_sl])
    plsc.subcore_barrier()

    for stage in range(NSTAGE):
        sbase = chunk_base + stage * STAGE_CHUNKS
        pltpu.sync_copy(dst_hbm.at[pl.ds(sbase, STAGE_CHUNKS)], idx_v)

        def scatter_pair(g, carry):
            pltpu.async_copy(
                msg_hbm.at[pl.ds((sbase + 2 * g) * E_CHUNK, 2 * E_CHUNK)],
                rows_v, gsem).wait()
            d0 = pltpu.async_copy(rows_v.at[pl.ds(0, E_CHUNK)],
                                  sp_buf.at[idx_v.at[2 * g]], gsem2,
                                  add=True)
            pltpu.sync_copy(rows_v.at[pl.ds(E_CHUNK, E_CHUNK)],
                            sp_buf.at[idx_v.at[2 * g + 1]], add=True)
            d0.wait()
            return carry

        lax.fori_loop(0, STAGE_CHUNKS // 2, scatter_pair, 0)

    plsc.subcore_barrier()
    pltpu.sync_copy(sp_buf.at[rows_sl], out_hbm.at[cid, rows_sl])


def _mlp_body(h_ref, p0_ref, p1_ref, w1_ref, b1_ref, w2_ref, b2_ref,
              sc_ref, sh_ref, out_ref):
    agg = h_ref[...] + p0_ref[...] + p1_ref[...]
    h1 = jnp.maximum(
        jnp.dot(agg, w1_ref[...], preferred_element_type=jnp.float32)
        + b1_ref[...], 0.0)
    h2 = (jnp.dot(h1, w2_ref[...], preferred_element_type=jnp.float32)
          + b2_ref[...])
    out_ref[...] = jnp.maximum(h2 * sc_ref[...] + sh_ref[...], 0.0)


def _mlp_final_body(h_ref, p0_ref, p1_ref, w1_ref, b1_ref, w2_ref, b2_ref,
                    sc_ref, sh_ref, wc_ref, bc_ref, out_ref, cls_ref):
    _mlp_body(h_ref, p0_ref, p1_ref, w1_ref, b1_ref, w2_ref, b2_ref,
              sc_ref, sh_ref, out_ref)
    cls_ref[...] = (jnp.dot(out_ref[...], wc_ref[...],
                            preferred_element_type=jnp.float32) + bc_ref[...])


_BLK = 1264
_row_spec = pl.BlockSpec((_BLK, D), lambda i: (i, 0))
_w_spec = pl.BlockSpec((D, D), lambda i: (0, 0))
_v_spec = pl.BlockSpec((1, D), lambda i: (0, 0))


def _tc_mlp(h, p0, p1, w1, b1, w2, b2, scale, shift):
    return pl.pallas_call(
        _mlp_body,
        grid=(NPAD // _BLK,),
        in_specs=[_row_spec, _row_spec, _row_spec, _w_spec, _v_spec,
                  _w_spec, _v_spec, _v_spec, _v_spec],
        out_specs=_row_spec,
        out_shape=jax.ShapeDtypeStruct((NPAD, D), jnp.float32),
    )(h, p0, p1, w1, b1, w2, b2, scale, shift)


def _tc_mlp_final(h, p0, p1, w1, b1, w2, b2, scale, shift, wc, bc):
    return pl.pallas_call(
        _mlp_final_body,
        grid=(NPAD // _BLK,),
        in_specs=[_row_spec, _row_spec, _row_spec, _w_spec, _v_spec,
                  _w_spec, _v_spec, _v_spec, _v_spec, _w_spec, _v_spec],
        out_specs=(_row_spec, _row_spec),
        out_shape=(jax.ShapeDtypeStruct((NPAD, D), jnp.float32),
                   jax.ShapeDtypeStruct((NPAD, D), jnp.float32)),
    )(h, p0, p1, w1, b1, w2, b2, scale, shift, wc, bc)


def kernel(x, edge_index, params):
    ei = edge_index.astype(jnp.int32)
    pad_e = EPAD - N_EDGES
    src = jnp.concatenate(
        [ei[0], jnp.full((pad_e,), N_NODES, dtype=jnp.int32)]
    ).reshape(EPAD // E_CHUNK, E_CHUNK)
    dst = jnp.concatenate(
        [ei[1], jnp.full((pad_e,), N_NODES, dtype=jnp.int32)]
    ).reshape(EPAD // E_CHUNK, E_CHUNK)

    h = jnp.zeros((NPAD, D), jnp.float32).at[:N_NODES].set(x)
    zeros = jnp.zeros((ROWS_PER_TILE, D), jnp.float32)

    for i in range(NUM_LAYERS):
        cp = params[f'conv{i}']
        bn = params[f'bn{i}']
        scale = (bn['gamma'] * lax.rsqrt(bn['var'] + BN_EPS)).reshape(1, D)
        shift = (bn['beta'] - bn['mean'] * scale[0]).reshape(1, D)
        b1 = cp['b1'].reshape(1, D)
        b2 = cp['b2'].reshape(1, D)

        parts, _ = _sc_segment_sum(src, dst, h, zeros)
        if i < NUM_LAYERS - 1:
            h = _tc_mlp(h, parts[0], parts[1], cp['W1'], b1,
                        cp['W2'], b2, scale, shift)
        else:
            wc = jnp.zeros((D, D), jnp.float32).at[:, :OUT_DIM].set(
                params['Wc'])
            bc = jnp.zeros((1, D), jnp.float32).at[0, :OUT_DIM].set(
                params['bc'])
            h, cls = _tc_mlp_final(h, parts[0], parts[1], cp['W1'], b1,
                                   cp['W2'], b2, scale, shift, wc, bc)
    return cls[:N_NODES, :OUT_DIM]


# speedup vs baseline: 2.4486x; 1.3724x over previous
import functools

import jax
import jax.numpy as jnp
from jax import lax
from jax.experimental import pallas as pl
from jax.experimental.pallas import tpu as pltpu
from jax.experimental.pallas import tpu_sc as plsc

N_NODES = 10000
D = 128
OUT_DIM = 32
NUM_LAYERS = 3
BN_EPS = 1e-5

NC = 2
NS = 16
NW = NC * NS

NPAD = 10112
ROWS_PER_TILE = NPAD // NS

E_CHUNK = 128
N_EDGES = 320000
EPW_CHUNKS = 80
NSTAGE = 2
STAGE_CHUNKS = EPW_CHUNKS // NSTAGE
EPW = EPW_CHUNKS * E_CHUNK
EPAD = EPW * NW

_sc_mesh = plsc.VectorSubcoreMesh(core_axis_name="c", subcore_axis_name="s")


@functools.partial(
    pl.kernel,
    mesh=_sc_mesh,
    out_type=(jax.ShapeDtypeStruct((NC, NPAD, D), jnp.float32),
              jax.ShapeDtypeStruct((EPAD, D), jnp.float32)),
    scratch_types=[
        pltpu.VMEM((STAGE_CHUNKS, E_CHUNK), jnp.int32),
        pltpu.VMEM((2 * E_CHUNK, D), jnp.float32),
        pltpu.VMEM_SHARED((NPAD, D), jnp.float32),
        pltpu.SemaphoreType.DMA,
        pltpu.SemaphoreType.DMA,
    ],
)
def _sc_segment_sum(src_hbm, dst_hbm, h_hbm, zeros_hbm, out_hbm, msg_hbm,
                    idx_v, rows_v, sp_buf, gsem, gsem2):
    cid = lax.axis_index("c")
    sid = lax.axis_index("s")
    wid = sid * NC + cid
    chunk_base = wid * EPW_CHUNKS
    row_base = sid * ROWS_PER_TILE
    rows_sl = pl.ds(row_base, ROWS_PER_TILE)

    pltpu.sync_copy(h_hbm.at[rows_sl], sp_buf.at[rows_sl])
    plsc.subcore_barrier()

    for stage in range(NSTAGE):
        sbase = chunk_base + stage * STAGE_CHUNKS
        pltpu.sync_copy(src_hbm.at[pl.ds(sbase, STAGE_CHUNKS)], idx_v)

        half = (rows_v.at[pl.ds(0, E_CHUNK)], rows_v.at[pl.ds(E_CHUNK, E_CHUNK)])
        sem = (gsem, gsem2)

        def _wait_gather(b):
            pltpu.make_async_copy(sp_buf.at[pl.ds(0, E_CHUNK)], half[b],
                                  sem[b]).wait()

        def _issue_gather(j, b):
            pltpu.async_copy(sp_buf.at[idx_v.at[j]], half[b], sem[b])

        def _write_msg(j, b):
            pltpu.sync_copy(half[b],
                            msg_hbm.at[pl.ds((sbase + j) * E_CHUNK, E_CHUNK)])

        _issue_gather(0, 0)
        _issue_gather(1, 1)

        def gather_pair(g, carry):
            j = 2 * g
            for b in range(2):
                _wait_gather(b)
                _write_msg(j + b, b)
                _issue_gather(j + b + 2, b)
            return carry

        lax.fori_loop(0, STAGE_CHUNKS // 2 - 1, gather_pair, 0)
        for b in range(2):
            _wait_gather(b)
            _write_msg(STAGE_CHUNKS - 2 + b, b)

    plsc.subcore_barrier()
    pltpu.sync_copy(zeros_hbm, sp_buf.at[rows_sl])
    plsc.subcore_barrier()

    for stage in range(NSTAGE):
        sbase = chunk_base + stage * STAGE_CHUNKS
        pltpu.sync_copy(dst_hbm.at[pl.ds(sbase, STAGE_CHUNKS)], idx_v)

        half = (rows_v.at[pl.ds(0, E_CHUNK)], rows_v.at[pl.ds(E_CHUNK, E_CHUNK)])
        sem = (gsem, gsem2)

        def _wait_read(b):
            pltpu.make_async_copy(msg_hbm.at[pl.ds(0, E_CHUNK)], half[b],
                                  sem[b]).wait()

        def _issue_read(j, b):
            pltpu.async_copy(
                msg_hbm.at[pl.ds((sbase + j) * E_CHUNK, E_CHUNK)], half[b],
                sem[b])

        def _scatter(j, b):
            pltpu.sync_copy(half[b], sp_buf.at[idx_v.at[j]], add=True)

        _issue_read(0, 0)
        _issue_read(1, 1)

        def scatter_pair(g, carry):
            j = 2 * g
            for b in range(2):
                _wait_read(b)
                _scatter(j + b, b)
                _issue_read(j + b + 2, b)
            return carry

        lax.fori_loop(0, STAGE_CHUNKS // 2 - 1, scatter_pair, 0)
        for b in range(2):
            _wait_read(b)
            _scatter(STAGE_CHUNKS - 2 + b, b)

    plsc.subcore_barrier()
    pltpu.sync_copy(sp_buf.at[rows_sl], out_hbm.at[cid, rows_sl])


def _mlp_body(h_ref, p0_ref, p1_ref, w1_ref, b1_ref, w2_ref, b2_ref,
              sc_ref, sh_ref, out_ref):
    agg = h_ref[...] + p0_ref[...] + p1_ref[...]
    h1 = jnp.maximum(
        jnp.dot(agg, w1_ref[...], preferred_element_type=jnp.float32)
        + b1_ref[...], 0.0)
    h2 = (jnp.dot(h1, w2_ref[...], preferred_element_type=jnp.float32)
          + b2_ref[...])
    out_ref[...] = jnp.maximum(h2 * sc_ref[...] + sh_ref[...], 0.0)


def _mlp_final_body(h_ref, p0_ref, p1_ref, w1_ref, b1_ref, w2_ref, b2_ref,
                    sc_ref, sh_ref, wc_ref, bc_ref, out_ref, cls_ref):
    _mlp_body(h_ref, p0_ref, p1_ref, w1_ref, b1_ref, w2_ref, b2_ref,
              sc_ref, sh_ref, out_ref)
    cls_ref[...] = (jnp.dot(out_ref[...], wc_ref[...],
                            preferred_element_type=jnp.float32) + bc_ref[...])


_BLK = 1264
_row_spec = pl.BlockSpec((_BLK, D), lambda i: (i, 0))
_w_spec = pl.BlockSpec((D, D), lambda i: (0, 0))
_v_spec = pl.BlockSpec((1, D), lambda i: (0, 0))


def _tc_mlp(h, p0, p1, w1, b1, w2, b2, scale, shift):
    return pl.pallas_call(
        _mlp_body,
        grid=(NPAD // _BLK,),
        in_specs=[_row_spec, _row_spec, _row_spec, _w_spec, _v_spec,
                  _w_spec, _v_spec, _v_spec, _v_spec],
        out_specs=_row_spec,
        out_shape=jax.ShapeDtypeStruct((NPAD, D), jnp.float32),
    )(h, p0, p1, w1, b1, w2, b2, scale, shift)


def _tc_mlp_final(h, p0, p1, w1, b1, w2, b2, scale, shift, wc, bc):
    return pl.pallas_call(
        _mlp_final_body,
        grid=(NPAD // _BLK,),
        in_specs=[_row_spec, _row_spec, _row_spec, _w_spec, _v_spec,
                  _w_spec, _v_spec, _v_spec, _v_spec, _w_spec, _v_spec],
        out_specs=(_row_spec, _row_spec),
        out_shape=(jax.ShapeDtypeStruct((NPAD, D), jnp.float32),
                   jax.ShapeDtypeStruct((NPAD, D), jnp.float32)),
    )(h, p0, p1, w1, b1, w2, b2, scale, shift, wc, bc)


def kernel(x, edge_index, params):
    ei = edge_index.astype(jnp.int32)
    pad_e = EPAD - N_EDGES
    src = jnp.concatenate(
        [ei[0], jnp.full((pad_e,), N_NODES, dtype=jnp.int32)]
    ).reshape(EPAD // E_CHUNK, E_CHUNK)
    dst = jnp.concatenate(
        [ei[1], jnp.full((pad_e,), N_NODES, dtype=jnp.int32)]
    ).reshape(EPAD // E_CHUNK, E_CHUNK)

    h = jnp.zeros((NPAD, D), jnp.float32).at[:N_NODES].set(x)
    zeros = jnp.zeros((ROWS_PER_TILE, D), jnp.float32)

    for i in range(NUM_LAYERS):
        cp = params[f'conv{i}']
        bn = params[f'bn{i}']
        scale = (bn['gamma'] * lax.rsqrt(bn['var'] + BN_EPS)).reshape(1, D)
        shift = (bn['beta'] - bn['mean'] * scale[0]).reshape(1, D)
        b1 = cp['b1'].reshape(1, D)
        b2 = cp['b2'].reshape(1, D)

        parts, _ = _sc_segment_sum(src, dst, h, zeros)
        if i < NUM_LAYERS - 1:
            h = _tc_mlp(h, parts[0], parts[1], cp['W1'], b1,
                        cp['W2'], b2, scale, shift)
        else:
            wc = jnp.zeros((D, D), jnp.float32).at[:, :OUT_DIM].set(
                params['Wc'])
            bc = jnp.zeros((1, D), jnp.float32).at[0, :OUT_DIM].set(
                params['bc'])
            h, cls = _tc_mlp_final(h, parts[0], parts[1], cp['W1'], b1,
                                   cp['W2'], b2, scale, shift, wc, bc)
    return cls[:N_NODES, :OUT_DIM]


# speedup vs baseline: 2.4691x; 1.0084x over previous
import functools

import jax
import jax.numpy as jnp
from jax import lax
from jax.experimental import pallas as pl
from jax.experimental.pallas import tpu as pltpu
from jax.experimental.pallas import tpu_sc as plsc

N_NODES = 10000
D = 128
OUT_DIM = 32
NUM_LAYERS = 3
BN_EPS = 1e-5

NC = 2
NS = 16
NW = NC * NS

NPAD = 10112
ROWS_PER_TILE = NPAD // NS

E_CHUNK = 128
N_EDGES = 320000
EPW_CHUNKS = 80
NSTAGE = 2
STAGE_CHUNKS = EPW_CHUNKS // NSTAGE
EPW = EPW_CHUNKS * E_CHUNK
EPAD = EPW * NW

_sc_mesh = plsc.VectorSubcoreMesh(core_axis_name="c", subcore_axis_name="s")


@functools.partial(
    pl.kernel,
    mesh=_sc_mesh,
    out_type=(jax.ShapeDtypeStruct((NC, NPAD, D), jnp.float32),
              jax.ShapeDtypeStruct((EPAD, D), jnp.float32)),
    scratch_types=[
        pltpu.VMEM((STAGE_CHUNKS, E_CHUNK), jnp.int32),
        pltpu.VMEM((2 * E_CHUNK, D), jnp.float32),
        pltpu.VMEM_SHARED((NPAD, D), jnp.float32),
        pltpu.SemaphoreType.DMA,
        pltpu.SemaphoreType.DMA,
    ],
)
def _sc_segment_sum(src_hbm, dst_hbm, h_hbm, zeros_hbm, out_hbm, msg_hbm,
                    idx_v, rows_v, sp_buf, gsem, gsem2):
    cid = lax.axis_index("c")
    sid = lax.axis_index("s")
    wid = sid * NC + cid
    chunk_base = wid * EPW_CHUNKS
    row_base = sid * ROWS_PER_TILE
    rows_sl = pl.ds(row_base, ROWS_PER_TILE)

    pltpu.sync_copy(h_hbm.at[rows_sl], sp_buf.at[rows_sl])
    plsc.subcore_barrier()

    for stage in range(NSTAGE):
        sbase = chunk_base + stage * STAGE_CHUNKS
        pltpu.sync_copy(src_hbm.at[pl.ds(sbase, STAGE_CHUNKS)], idx_v)

        half = (rows_v.at[pl.ds(0, E_CHUNK)], rows_v.at[pl.ds(E_CHUNK, E_CHUNK)])
        sem = (gsem, gsem2)

        def _wait_gather(b):
            pltpu.make_async_copy(sp_buf.at[pl.ds(0, E_CHUNK)], half[b],
                                  sem[b]).wait()

        def _issue_gather(j, b):
            pltpu.async_copy(sp_buf.at[idx_v.at[j]], half[b], sem[b])

        def _write_msg(j, b):
            pltpu.sync_copy(half[b],
                            msg_hbm.at[pl.ds((sbase + j) * E_CHUNK, E_CHUNK)])

        _issue_gather(0, 0)
        _issue_gather(1, 1)

        def gather_pair(g, carry):
            j = 2 * g
            for b in range(2):
                _wait_gather(b)
                _write_msg(j + b, b)
                _issue_gather(j + b + 2, b)
            return carry

        lax.fori_loop(0, STAGE_CHUNKS // 2 - 1, gather_pair, 0)
        for b in range(2):
            _wait_gather(b)
            _write_msg(STAGE_CHUNKS - 2 + b, b)

    plsc.subcore_barrier()
    @pl.when(cid == 1)
    def _zero():
        pltpu.sync_copy(zeros_hbm, sp_buf.at[rows_sl])
    plsc.subcore_barrier()

    for stage in range(NSTAGE):
        sbase = chunk_base + stage * STAGE_CHUNKS
        pltpu.sync_copy(dst_hbm.at[pl.ds(sbase, STAGE_CHUNKS)], idx_v)

        half = (rows_v.at[pl.ds(0, E_CHUNK)], rows_v.at[pl.ds(E_CHUNK, E_CHUNK)])
        sem = (gsem, gsem2)

        def _wait_read(b):
            pltpu.make_async_copy(msg_hbm.at[pl.ds(0, E_CHUNK)], half[b],
                                  sem[b]).wait()

        def _issue_read(j, b):
            pltpu.async_copy(
                msg_hbm.at[pl.ds((sbase + j) * E_CHUNK, E_CHUNK)], half[b],
                sem[b])

        def _scatter(j, b):
            pltpu.sync_copy(half[b], sp_buf.at[idx_v.at[j]], add=True)

        _issue_read(0, 0)
        _issue_read(1, 1)

        def scatter_pair(g, carry):
            j = 2 * g
            for b in range(2):
                _wait_read(b)
                _scatter(j + b, b)
                _issue_read(j + b + 2, b)
            return carry

        lax.fori_loop(0, STAGE_CHUNKS // 2 - 1, scatter_pair, 0)
        for b in range(2):
            _wait_read(b)
            _scatter(STAGE_CHUNKS - 2 + b, b)

    plsc.subcore_barrier()
    pltpu.sync_copy(sp_buf.at[rows_sl], out_hbm.at[cid, rows_sl])


def _mlp_body(p0_ref, p1_ref, w1_ref, b1_ref, w2_ref, b2_ref,
              sc_ref, sh_ref, out_ref):
    agg = p0_ref[...] + p1_ref[...]
    h1 = jnp.maximum(
        jnp.dot(agg, w1_ref[...], preferred_element_type=jnp.float32)
        + b1_ref[...], 0.0)
    h2 = (jnp.dot(h1, w2_ref[...], preferred_element_type=jnp.float32)
          + b2_ref[...])
    out_ref[...] = jnp.maximum(h2 * sc_ref[...] + sh_ref[...], 0.0)


def _mlp_final_body(p0_ref, p1_ref, w1_ref, b1_ref, w2_ref, b2_ref,
                    sc_ref, sh_ref, wc_ref, bc_ref, out_ref, cls_ref):
    _mlp_body(p0_ref, p1_ref, w1_ref, b1_ref, w2_ref, b2_ref,
              sc_ref, sh_ref, out_ref)
    cls_ref[...] = (jnp.dot(out_ref[...], wc_ref[...],
                            preferred_element_type=jnp.float32) + bc_ref[...])


_BLK = 1264
_row_spec = pl.BlockSpec((_BLK, D), lambda i: (i, 0))
_w_spec = pl.BlockSpec((D, D), lambda i: (0, 0))
_v_spec = pl.BlockSpec((1, D), lambda i: (0, 0))


def _tc_mlp(p0, p1, w1, b1, w2, b2, scale, shift):
    return pl.pallas_call(
        _mlp_body,
        grid=(NPAD // _BLK,),
        in_specs=[_row_spec, _row_spec, _w_spec, _v_spec,
                  _w_spec, _v_spec, _v_spec, _v_spec],
        out_specs=_row_spec,
        out_shape=jax.ShapeDtypeStruct((NPAD, D), jnp.float32),
    )(p0, p1, w1, b1, w2, b2, scale, shift)


def _tc_mlp_final(p0, p1, w1, b1, w2, b2, scale, shift, wc, bc):
    return pl.pallas_call(
        _mlp_final_body,
        grid=(NPAD // _BLK,),
        in_specs=[_row_spec, _row_spec, _w_spec, _v_spec,
                  _w_spec, _v_spec, _v_spec, _v_spec, _w_spec, _v_spec],
        out_specs=(_row_spec, _row_spec),
        out_shape=(jax.ShapeDtypeStruct((NPAD, D), jnp.float32),
                   jax.ShapeDtypeStruct((NPAD, D), jnp.float32)),
    )(p0, p1, w1, b1, w2, b2, scale, shift, wc, bc)


def kernel(x, edge_index, params):
    ei = edge_index.astype(jnp.int32)
    pad_e = EPAD - N_EDGES
    src = jnp.concatenate(
        [ei[0], jnp.full((pad_e,), N_NODES, dtype=jnp.int32)]
    ).reshape(EPAD // E_CHUNK, E_CHUNK)
    dst = jnp.concatenate(
        [ei[1], jnp.full((pad_e,), N_NODES, dtype=jnp.int32)]
    ).reshape(EPAD // E_CHUNK, E_CHUNK)

    h = jnp.zeros((NPAD, D), jnp.float32).at[:N_NODES].set(x)
    zeros = jnp.zeros((ROWS_PER_TILE, D), jnp.float32)

    for i in range(NUM_LAYERS):
        cp = params[f'conv{i}']
        bn = params[f'bn{i}']
        scale = (bn['gamma'] * lax.rsqrt(bn['var'] + BN_EPS)).reshape(1, D)
        shift = (bn['beta'] - bn['mean'] * scale[0]).reshape(1, D)
        b1 = cp['b1'].reshape(1, D)
        b2 = cp['b2'].reshape(1, D)

        parts, _ = _sc_segment_sum(src, dst, h, zeros)
        if i < NUM_LAYERS - 1:
            h = _tc_mlp(parts[0], parts[1], cp['W1'], b1,
                        cp['W2'], b2, scale, shift)
        else:
            wc = jnp.zeros((D, D), jnp.float32).at[:, :OUT_DIM].set(
                params['Wc'])
            bc = jnp.zeros((1, D), jnp.float32).at[0, :OUT_DIM].set(
                params['bc'])
            h, cls = _tc_mlp_final(parts[0], parts[1], cp['W1'], b1,
                                   cp['W2'], b2, scale, shift, wc, bc)
    return cls[:N_NODES, :OUT_DIM]


# speedup vs baseline: 2.5180x; 1.0198x over previous
import functools

import jax
import jax.numpy as jnp
from jax import lax
from jax.experimental import pallas as pl
from jax.experimental.pallas import tpu as pltpu
from jax.experimental.pallas import tpu_sc as plsc

N_NODES = 10000
D = 128
OUT_DIM = 32
NUM_LAYERS = 3
BN_EPS = 1e-5

NC = 2
NS = 16
NW = NC * NS

NPAD = 10112
ROWS_PER_TILE = NPAD // NS

E_CHUNK = 128
N_EDGES = 320000
EPW_CHUNKS = 80
NSTAGE = 1
STAGE_CHUNKS = EPW_CHUNKS // NSTAGE
EPW = EPW_CHUNKS * E_CHUNK
EPAD = EPW * NW

_sc_mesh = plsc.VectorSubcoreMesh(core_axis_name="c", subcore_axis_name="s")


@functools.partial(
    pl.kernel,
    mesh=_sc_mesh,
    out_type=(jax.ShapeDtypeStruct((NC, NPAD, D), jnp.float32),
              jax.ShapeDtypeStruct((EPAD, D), jnp.float32)),
    scratch_types=[
        pltpu.VMEM((STAGE_CHUNKS, E_CHUNK), jnp.int32),
        pltpu.VMEM((2 * E_CHUNK, D), jnp.float32),
        pltpu.VMEM_SHARED((NPAD, D), jnp.float32),
        pltpu.SemaphoreType.DMA,
        pltpu.SemaphoreType.DMA,
    ],
)
def _sc_segment_sum(src_hbm, dst_hbm, h_hbm, zeros_hbm, out_hbm, msg_hbm,
                    idx_v, rows_v, sp_buf, gsem, gsem2):
    cid = lax.axis_index("c")
    sid = lax.axis_index("s")
    wid = sid * NC + cid
    chunk_base = wid * EPW_CHUNKS
    row_base = sid * ROWS_PER_TILE
    rows_sl = pl.ds(row_base, ROWS_PER_TILE)

    pltpu.sync_copy(h_hbm.at[rows_sl], sp_buf.at[rows_sl])
    plsc.subcore_barrier()

    for stage in range(NSTAGE):
        sbase = chunk_base + stage * STAGE_CHUNKS
        pltpu.sync_copy(src_hbm.at[pl.ds(sbase, STAGE_CHUNKS)], idx_v)

        half = (rows_v.at[pl.ds(0, E_CHUNK)], rows_v.at[pl.ds(E_CHUNK, E_CHUNK)])
        sem = (gsem, gsem2)

        def _wait_gather(b):
            pltpu.make_async_copy(sp_buf.at[pl.ds(0, E_CHUNK)], half[b],
                                  sem[b]).wait()

        def _issue_gather(j, b):
            pltpu.async_copy(sp_buf.at[idx_v.at[j]], half[b], sem[b])

        def _write_msg(j, b):
            pltpu.sync_copy(half[b],
                            msg_hbm.at[pl.ds((sbase + j) * E_CHUNK, E_CHUNK)])

        _issue_gather(0, 0)
        _issue_gather(1, 1)

        def gather_pair(g, carry):
            j = 2 * g
            for b in range(2):
                _wait_gather(b)
                _write_msg(j + b, b)
                _issue_gather(j + b + 2, b)
            return carry

        lax.fori_loop(0, STAGE_CHUNKS // 2 - 1, gather_pair, 0)
        for b in range(2):
            _wait_gather(b)
            _write_msg(STAGE_CHUNKS - 2 + b, b)

    plsc.subcore_barrier()
    @pl.when(cid == 1)
    def _zero():
        pltpu.sync_copy(zeros_hbm, sp_buf.at[rows_sl])
    plsc.subcore_barrier()

    for stage in range(NSTAGE):
        sbase = chunk_base + stage * STAGE_CHUNKS
        pltpu.sync_copy(dst_hbm.at[pl.ds(sbase, STAGE_CHUNKS)], idx_v)

        half = (rows_v.at[pl.ds(0, E_CHUNK)], rows_v.at[pl.ds(E_CHUNK, E_CHUNK)])
        sem = (gsem, gsem2)

        def _wait_read(b):
            pltpu.make_async_copy(msg_hbm.at[pl.ds(0, E_CHUNK)], half[b],
                                  sem[b]).wait()

        def _issue_read(j, b):
            pltpu.async_copy(
                msg_hbm.at[pl.ds((sbase + j) * E_CHUNK, E_CHUNK)], half[b],
                sem[b])

        def _scatter(j, b):
            pltpu.sync_copy(half[b], sp_buf.at[idx_v.at[j]], add=True)

        _issue_read(0, 0)
        _issue_read(1, 1)

        def scatter_pair(g, carry):
            j = 2 * g
            for b in range(2):
                _wait_read(b)
                _scatter(j + b, b)
                _issue_read(j + b + 2, b)
            return carry

        lax.fori_loop(0, STAGE_CHUNKS // 2 - 1, scatter_pair, 0)
        for b in range(2):
            _wait_read(b)
            _scatter(STAGE_CHUNKS - 2 + b, b)

    plsc.subcore_barrier()
    pltpu.sync_copy(sp_buf.at[rows_sl], out_hbm.at[cid, rows_sl])


def _mlp_body(p0_ref, p1_ref, w1_ref, b1_ref, w2_ref, b2_ref,
              sc_ref, sh_ref, out_ref):
    agg = p0_ref[...] + p1_ref[...]
    h1 = jnp.maximum(
        jnp.dot(agg, w1_ref[...], preferred_element_type=jnp.float32)
        + b1_ref[...], 0.0)
    h2 = (jnp.dot(h1, w2_ref[...], preferred_element_type=jnp.float32)
          + b2_ref[...])
    out_ref[...] = jnp.maximum(h2 * sc_ref[...] + sh_ref[...], 0.0)


def _mlp_final_body(p0_ref, p1_ref, w1_ref, b1_ref, w2_ref, b2_ref,
                    sc_ref, sh_ref, wc_ref, bc_ref, out_ref, cls_ref):
    _mlp_body(p0_ref, p1_ref, w1_ref, b1_ref, w2_ref, b2_ref,
              sc_ref, sh_ref, out_ref)
    cls_ref[...] = (jnp.dot(out_ref[...], wc_ref[...],
                            preferred_element_type=jnp.float32) + bc_ref[...])


_BLK = 1264
_row_spec = pl.BlockSpec((_BLK, D), lambda i: (i, 0))
_w_spec = pl.BlockSpec((D, D), lambda i: (0, 0))
_v_spec = pl.BlockSpec((1, D), lambda i: (0, 0))


def _tc_mlp(p0, p1, w1, b1, w2, b2, scale, shift):
    return pl.pallas_call(
        _mlp_body,
        grid=(NPAD // _BLK,),
        in_specs=[_row_spec, _row_spec, _w_spec, _v_spec,
                  _w_spec, _v_spec, _v_spec, _v_spec],
        out_specs=_row_spec,
        out_shape=jax.ShapeDtypeStruct((NPAD, D), jnp.float32),
    )(p0, p1, w1, b1, w2, b2, scale, shift)


def _tc_mlp_final(p0, p1, w1, b1, w2, b2, scale, shift, wc, bc):
    return pl.pallas_call(
        _mlp_final_body,
        grid=(NPAD // _BLK,),
        in_specs=[_row_spec, _row_spec, _w_spec, _v_spec,
                  _w_spec, _v_spec, _v_spec, _v_spec, _w_spec, _v_spec],
        out_specs=(_row_spec, _row_spec),
        out_shape=(jax.ShapeDtypeStruct((NPAD, D), jnp.float32),
                   jax.ShapeDtypeStruct((NPAD, D), jnp.float32)),
    )(p0, p1, w1, b1, w2, b2, scale, shift, wc, bc)


def kernel(x, edge_index, params):
    ei = edge_index.astype(jnp.int32)
    pad_e = EPAD - N_EDGES
    src = jnp.concatenate(
        [ei[0], jnp.full((pad_e,), N_NODES, dtype=jnp.int32)]
    ).reshape(EPAD // E_CHUNK, E_CHUNK)
    dst = jnp.concatenate(
        [ei[1], jnp.full((pad_e,), N_NODES, dtype=jnp.int32)]
    ).reshape(EPAD // E_CHUNK, E_CHUNK)

    h = jnp.zeros((NPAD, D), jnp.float32).at[:N_NODES].set(x)
    zeros = jnp.zeros((ROWS_PER_TILE, D), jnp.float32)

    for i in range(NUM_LAYERS):
        cp = params[f'conv{i}']
        bn = params[f'bn{i}']
        scale = (bn['gamma'] * lax.rsqrt(bn['var'] + BN_EPS)).reshape(1, D)
        shift = (bn['beta'] - bn['mean'] * scale[0]).reshape(1, D)
        b1 = cp['b1'].reshape(1, D)
        b2 = cp['b2'].reshape(1, D)

        parts, _ = _sc_segment_sum(src, dst, h, zeros)
        if i < NUM_LAYERS - 1:
            h = _tc_mlp(parts[0], parts[1], cp['W1'], b1,
                        cp['W2'], b2, scale, shift)
        else:
            wc = jnp.zeros((D, D), jnp.float32).at[:, :OUT_DIM].set(
                params['Wc'])
            bc = jnp.zeros((1, D), jnp.float32).at[0, :OUT_DIM].set(
                params['bc'])
            h, cls = _tc_mlp_final(parts[0], parts[1], cp['W1'], b1,
                                   cp['W2'], b2, scale, shift, wc, bc)
    return cls[:N_NODES, :OUT_DIM]


# speedup vs baseline: 2.5785x; 1.0240x over previous
import functools

import jax
import jax.numpy as jnp
from jax import lax
from jax.experimental import pallas as pl
from jax.experimental.pallas import tpu as pltpu
from jax.experimental.pallas import tpu_sc as plsc

N_NODES = 10000
D = 128
OUT_DIM = 32
NUM_LAYERS = 3
BN_EPS = 1e-5

NC = 2
NS = 16
NW = NC * NS

NPAD = 10112
ROWS_PER_TILE = NPAD // NS

E_CHUNK = 128
N_EDGES = 320000
EPW_CHUNKS = 80
NSTAGE = 1
STAGE_CHUNKS = EPW_CHUNKS // NSTAGE
EPW = EPW_CHUNKS * E_CHUNK
EPAD = EPW * NW

_sc_mesh = plsc.VectorSubcoreMesh(core_axis_name="c", subcore_axis_name="s")


@functools.partial(
    pl.kernel,
    mesh=_sc_mesh,
    out_type=(jax.ShapeDtypeStruct((NC, NPAD, D), jnp.float32),
              jax.ShapeDtypeStruct((EPAD, D), jnp.float32)),
    scratch_types=[
        pltpu.VMEM((STAGE_CHUNKS, E_CHUNK), jnp.int32),
        pltpu.VMEM((2 * E_CHUNK, D), jnp.float32),
        pltpu.VMEM_SHARED((NPAD, D), jnp.float32),
        pltpu.SemaphoreType.DMA,
        pltpu.SemaphoreType.DMA,
    ],
)
def _sc_segment_sum(src_hbm, dst_hbm, h_hbm, zeros_hbm, out_hbm, msg_hbm,
                    idx_v, rows_v, sp_buf, gsem, gsem2):
    cid = lax.axis_index("c")
    sid = lax.axis_index("s")
    wid = sid * NC + cid
    chunk_base = wid * EPW_CHUNKS
    row_base = sid * ROWS_PER_TILE
    rows_sl = pl.ds(row_base, ROWS_PER_TILE)

    pltpu.sync_copy(h_hbm.at[rows_sl], sp_buf.at[rows_sl])
    plsc.subcore_barrier()

    for stage in range(NSTAGE):
        sbase = chunk_base + stage * STAGE_CHUNKS
        pltpu.sync_copy(src_hbm.at[pl.ds(sbase, STAGE_CHUNKS)], idx_v)

        half = (rows_v.at[pl.ds(0, E_CHUNK)], rows_v.at[pl.ds(E_CHUNK, E_CHUNK)])
        sem = (gsem, gsem2)

        def _wait_gather(b):
            pltpu.make_async_copy(sp_buf.at[pl.ds(0, E_CHUNK)], half[b],
                                  sem[b]).wait()

        def _issue_gather(j, b):
            pltpu.async_copy(sp_buf.at[idx_v.at[j]], half[b], sem[b])

        def _write_msg(j, b):
            pltpu.sync_copy(half[b],
                            msg_hbm.at[pl.ds((sbase + j) * E_CHUNK, E_CHUNK)])

        _issue_gather(0, 0)
        _issue_gather(1, 1)

        def gather_pair(g, carry):
            j = 2 * g
            for b in range(2):
                _wait_gather(b)
                _write_msg(j + b, b)
                _issue_gather(j + b + 2, b)
            return carry

        lax.fori_loop(0, STAGE_CHUNKS // 2 - 1, gather_pair, 0)
        for b in range(2):
            _wait_gather(b)
            _write_msg(STAGE_CHUNKS - 2 + b, b)

    plsc.subcore_barrier()
    @pl.when((cid == 0) != (sid < NS // 2))
    def _zero():
        pltpu.sync_copy(zeros_hbm, sp_buf.at[rows_sl])
    plsc.subcore_barrier()

    for stage in range(NSTAGE):
        sbase = chunk_base + stage * STAGE_CHUNKS
        pltpu.sync_copy(dst_hbm.at[pl.ds(sbase, STAGE_CHUNKS)], idx_v)

        half = (rows_v.at[pl.ds(0, E_CHUNK)], rows_v.at[pl.ds(E_CHUNK, E_CHUNK)])
        sem = (gsem, gsem2)

        def _wait_read(b):
            pltpu.make_async_copy(msg_hbm.at[pl.ds(0, E_CHUNK)], half[b],
                                  sem[b]).wait()

        def _issue_read(j, b):
            pltpu.async_copy(
                msg_hbm.at[pl.ds((sbase + j) * E_CHUNK, E_CHUNK)], half[b],
                sem[b])

        def _scatter(j, b):
            pltpu.sync_copy(half[b], sp_buf.at[idx_v.at[j]], add=True)

        _issue_read(0, 0)
        _issue_read(1, 1)

        def scatter_pair(g, carry):
            j = 2 * g
            for b in range(2):
                _wait_read(b)
                _scatter(j + b, b)
                _issue_read(j + b + 2, b)
            return carry

        lax.fori_loop(0, STAGE_CHUNKS // 2 - 1, scatter_pair, 0)
        for b in range(2):
            _wait_read(b)
            _scatter(STAGE_CHUNKS - 2 + b, b)

    plsc.subcore_barrier()
    pltpu.sync_copy(sp_buf.at[rows_sl], out_hbm.at[cid, rows_sl])


def _mlp_body(p0_ref, p1_ref, w1_ref, b1_ref, w2_ref, b2_ref,
              sc_ref, sh_ref, out_ref):
    agg = p0_ref[...] + p1_ref[...]
    h1 = jnp.maximum(
        jnp.dot(agg, w1_ref[...], preferred_element_type=jnp.float32)
        + b1_ref[...], 0.0)
    h2 = (jnp.dot(h1, w2_ref[...], preferred_element_type=jnp.float32)
          + b2_ref[...])
    out_ref[...] = jnp.maximum(h2 * sc_ref[...] + sh_ref[...], 0.0)


def _mlp_final_body(p0_ref, p1_ref, w1_ref, b1_ref, w2_ref, b2_ref,
                    sc_ref, sh_ref, wc_ref, bc_ref, out_ref, cls_ref):
    _mlp_body(p0_ref, p1_ref, w1_ref, b1_ref, w2_ref, b2_ref,
              sc_ref, sh_ref, out_ref)
    cls_ref[...] = (jnp.dot(out_ref[...], wc_ref[...],
                            preferred_element_type=jnp.float32) + bc_ref[...])


_BLK = 2528
_row_spec = pl.BlockSpec((_BLK, D), lambda i: (i, 0))
_w_spec = pl.BlockSpec((D, D), lambda i: (0, 0))
_v_spec = pl.BlockSpec((1, D), lambda i: (0, 0))


def _tc_mlp(p0, p1, w1, b1, w2, b2, scale, shift):
    return pl.pallas_call(
        _mlp_body,
        grid=(NPAD // _BLK,),
        in_specs=[_row_spec, _row_spec, _w_spec, _v_spec,
                  _w_spec, _v_spec, _v_spec, _v_spec],
        out_specs=_row_spec,
        out_shape=jax.ShapeDtypeStruct((NPAD, D), jnp.float32),
    )(p0, p1, w1, b1, w2, b2, scale, shift)


def _tc_mlp_final(p0, p1, w1, b1, w2, b2, scale, shift, wc, bc):
    return pl.pallas_call(
        _mlp_final_body,
        grid=(NPAD // _BLK,),
        in_specs=[_row_spec, _row_spec, _w_spec, _v_spec,
                  _w_spec, _v_spec, _v_spec, _v_spec, _w_spec, _v_spec],
        out_specs=(_row_spec, _row_spec),
        out_shape=(jax.ShapeDtypeStruct((NPAD, D), jnp.float32),
                   jax.ShapeDtypeStruct((NPAD, D), jnp.float32)),
    )(p0, p1, w1, b1, w2, b2, scale, shift, wc, bc)


def kernel(x, edge_index, params):
    ei = edge_index.astype(jnp.int32)
    pad_e = EPAD - N_EDGES
    src = jnp.concatenate(
        [ei[0], jnp.full((pad_e,), N_NODES, dtype=jnp.int32)]
    ).reshape(EPAD // E_CHUNK, E_CHUNK)
    dst = jnp.concatenate(
        [ei[1], jnp.full((pad_e,), N_NODES, dtype=jnp.int32)]
    ).reshape(EPAD // E_CHUNK, E_CHUNK)

    h = jnp.zeros((NPAD, D), jnp.float32).at[:N_NODES].set(x)
    zeros = jnp.zeros((ROWS_PER_TILE, D), jnp.float32)

    for i in range(NUM_LAYERS):
        cp = params[f'conv{i}']
        bn = params[f'bn{i}']
        scale = (bn['gamma'] * lax.rsqrt(bn['var'] + BN_EPS)).reshape(1, D)
        shift = (bn['beta'] - bn['mean'] * scale[0]).reshape(1, D)
        b1 = cp['b1'].reshape(1, D)
        b2 = cp['b2'].reshape(1, D)

        parts, _ = _sc_segment_sum(src, dst, h, zeros)
        if i < NUM_LAYERS - 1:
            h = _tc_mlp(parts[0], parts[1], cp['W1'], b1,
                        cp['W2'], b2, scale, shift)
        else:
            wc = jnp.zeros((D, D), jnp.float32).at[:, :OUT_DIM].set(
                params['Wc'])
            bc = jnp.zeros((1, D), jnp.float32).at[0, :OUT_DIM].set(
                params['bc'])
            h, cls = _tc_mlp_final(parts[0], parts[1], cp['W1'], b1,
                                   cp['W2'], b2, scale, shift, wc, bc)
    return cls[:N_NODES, :OUT_DIM]


# speedup vs baseline: 2.5973x; 1.0073x over previous
import functools

import jax
import jax.numpy as jnp
from jax import lax
from jax.experimental import pallas as pl
from jax.experimental.pallas import tpu as pltpu
from jax.experimental.pallas import tpu_sc as plsc

N_NODES = 10000
D = 128
OUT_DIM = 32
NUM_LAYERS = 3
BN_EPS = 1e-5

NC = 2
NS = 16
NW = NC * NS

NPAD = 10112
ROWS_PER_TILE = NPAD // NS

E_CHUNK = 128
N_EDGES = 320000
EPW_CHUNKS = 80
NSTAGE = 1
STAGE_CHUNKS = EPW_CHUNKS // NSTAGE
EPW = EPW_CHUNKS * E_CHUNK
EPAD = EPW * NW

_sc_mesh = plsc.VectorSubcoreMesh(core_axis_name="c", subcore_axis_name="s")


@functools.partial(
    pl.kernel,
    mesh=_sc_mesh,
    out_type=(jax.ShapeDtypeStruct((NC, NPAD, D), jnp.float32),
              jax.ShapeDtypeStruct((EPAD, D), jnp.float32)),
    scratch_types=[
        pltpu.VMEM((STAGE_CHUNKS, E_CHUNK), jnp.int32),
        pltpu.VMEM((2 * E_CHUNK, D), jnp.float32),
        pltpu.VMEM_SHARED((NPAD, D), jnp.float32),
        pltpu.SemaphoreType.DMA,
        pltpu.SemaphoreType.DMA,
    ],
)
def _sc_segment_sum(src_hbm, dst_hbm, h_hbm, zeros_hbm, out_hbm, msg_hbm,
                    idx_v, rows_v, sp_buf, gsem, gsem2):
    cid = lax.axis_index("c")
    sid = lax.axis_index("s")
    wid = sid * NC + cid
    chunk_base = wid * EPW_CHUNKS
    row_base = sid * ROWS_PER_TILE
    rows_sl = pl.ds(row_base, ROWS_PER_TILE)

    hstage = pltpu.async_copy(h_hbm.at[rows_sl], sp_buf.at[rows_sl], gsem)

    for stage in range(NSTAGE):
        sbase = chunk_base + stage * STAGE_CHUNKS
        pltpu.sync_copy(src_hbm.at[pl.ds(sbase, STAGE_CHUNKS)], idx_v)
        hstage.wait()
        plsc.subcore_barrier()

        half = (rows_v.at[pl.ds(0, E_CHUNK)], rows_v.at[pl.ds(E_CHUNK, E_CHUNK)])
        sem = (gsem, gsem2)

        def _wait_gather(b):
            pltpu.make_async_copy(sp_buf.at[pl.ds(0, E_CHUNK)], half[b],
                                  sem[b]).wait()

        def _issue_gather(j, b):
            pltpu.async_copy(sp_buf.at[idx_v.at[j]], half[b], sem[b])

        def _write_msg(j, b):
            pltpu.sync_copy(half[b],
                            msg_hbm.at[pl.ds((sbase + j) * E_CHUNK, E_CHUNK)])

        _issue_gather(0, 0)
        _issue_gather(1, 1)

        def gather_pair(g, carry):
            j = 2 * g
            for b in range(2):
                _wait_gather(b)
                _write_msg(j + b, b)
                _issue_gather(j + b + 2, b)
            return carry

        lax.fori_loop(0, STAGE_CHUNKS // 2 - 1, gather_pair, 0)
        for b in range(2):
            _wait_gather(b)
            _write_msg(STAGE_CHUNKS - 2 + b, b)

    plsc.subcore_barrier()
    @pl.when((cid == 0) != (sid < NS // 2))
    def _zero():
        pltpu.sync_copy(zeros_hbm, sp_buf.at[rows_sl])
    plsc.subcore_barrier()

    for stage in range(NSTAGE):
        sbase = chunk_base + stage * STAGE_CHUNKS
        pltpu.sync_copy(dst_hbm.at[pl.ds(sbase, STAGE_CHUNKS)], idx_v)

        half = (rows_v.at[pl.ds(0, E_CHUNK)], rows_v.at[pl.ds(E_CHUNK, E_CHUNK)])
        sem = (gsem, gsem2)

        def _wait_read(b):
            pltpu.make_async_copy(msg_hbm.at[pl.ds(0, E_CHUNK)], half[b],
                                  sem[b]).wait()

        def _issue_read(j, b):
            pltpu.async_copy(
                msg_hbm.at[pl.ds((sbase + j) * E_CHUNK, E_CHUNK)], half[b],
                sem[b])

        def _scatter(j, b):
            pltpu.sync_copy(half[b], sp_buf.at[idx_v.at[j]], add=True)

        _issue_read(0, 0)
        _issue_read(1, 1)

        def scatter_pair(g, carry):
            j = 2 * g
            for b in range(2):
                _wait_read(b)
                _scatter(j + b, b)
                _issue_read(j + b + 2, b)
            return carry

        lax.fori_loop(0, STAGE_CHUNKS // 2 - 1, scatter_pair, 0)
        for b in range(2):
            _wait_read(b)
            _scatter(STAGE_CHUNKS - 2 + b, b)

    plsc.subcore_barrier()
    pltpu.sync_copy(sp_buf.at[rows_sl], out_hbm.at[cid, rows_sl])


def _mlp_body(p0_ref, p1_ref, w1_ref, b1_ref, w2_ref, b2_ref,
              sc_ref, sh_ref, out_ref):
    agg = p0_ref[...] + p1_ref[...]
    h1 = jnp.maximum(
        jnp.dot(agg, w1_ref[...], preferred_element_type=jnp.float32)
        + b1_ref[...], 0.0)
    h2 = (jnp.dot(h1, w2_ref[...], preferred_element_type=jnp.float32)
          + b2_ref[...])
    out_ref[...] = jnp.maximum(h2 * sc_ref[...] + sh_ref[...], 0.0)


def _mlp_final_body(p0_ref, p1_ref, w1_ref, b1_ref, w2_ref, b2_ref,
                    sc_ref, sh_ref, wc_ref, bc_ref, out_ref, cls_ref):
    _mlp_body(p0_ref, p1_ref, w1_ref, b1_ref, w2_ref, b2_ref,
              sc_ref, sh_ref, out_ref)
    cls_ref[...] = (jnp.dot(out_ref[...], wc_ref[...],
                            preferred_element_type=jnp.float32) + bc_ref[...])


_BLK = 2528
_row_spec = pl.BlockSpec((_BLK, D), lambda i: (i, 0))
_w_spec = pl.BlockSpec((D, D), lambda i: (0, 0))
_v_spec = pl.BlockSpec((1, D), lambda i: (0, 0))


def _tc_mlp(p0, p1, w1, b1, w2, b2, scale, shift):
    return pl.pallas_call(
        _mlp_body,
        grid=(NPAD // _BLK,),
        in_specs=[_row_spec, _row_spec, _w_spec, _v_spec,
                  _w_spec, _v_spec, _v_spec, _v_spec],
        out_specs=_row_spec,
        out_shape=jax.ShapeDtypeStruct((NPAD, D), jnp.float32),
    )(p0, p1, w1, b1, w2, b2, scale, shift)


def _tc_mlp_final(p0, p1, w1, b1, w2, b2, scale, shift, wc, bc):
    return pl.pallas_call(
        _mlp_final_body,
        grid=(NPAD // _BLK,),
        in_specs=[_row_spec, _row_spec, _w_spec, _v_spec,
                  _w_spec, _v_spec, _v_spec, _v_spec, _w_spec, _v_spec],
        out_specs=(_row_spec, _row_spec),
        out_shape=(jax.ShapeDtypeStruct((NPAD, D), jnp.float32),
                   jax.ShapeDtypeStruct((NPAD, D), jnp.float32)),
    )(p0, p1, w1, b1, w2, b2, scale, shift, wc, bc)


def kernel(x, edge_index, params):
    ei = edge_index.astype(jnp.int32)
    pad_e = EPAD - N_EDGES
    src = jnp.concatenate(
        [ei[0], jnp.full((pad_e,), N_NODES, dtype=jnp.int32)]
    ).reshape(EPAD // E_CHUNK, E_CHUNK)
    dst = jnp.concatenate(
        [ei[1], jnp.full((pad_e,), N_NODES, dtype=jnp.int32)]
    ).reshape(EPAD // E_CHUNK, E_CHUNK)

    h = jnp.zeros((NPAD, D), jnp.float32).at[:N_NODES].set(x)
    zeros = jnp.zeros((ROWS_PER_TILE, D), jnp.float32)

    for i in range(NUM_LAYERS):
        cp = params[f'conv{i}']
        bn = params[f'bn{i}']
        scale = (bn['gamma'] * lax.rsqrt(bn['var'] + BN_EPS)).reshape(1, D)
        shift = (bn['beta'] - bn['mean'] * scale[0]).reshape(1, D)
        b1 = cp['b1'].reshape(1, D)
        b2 = cp['b2'].reshape(1, D)

        parts, _ = _sc_segment_sum(src, dst, h, zeros)
        if i < NUM_LAYERS - 1:
            h = _tc_mlp(parts[0], parts[1], cp['W1'], b1,
                        cp['W2'], b2, scale, shift)
        else:
            wc = jnp.zeros((D, D), jnp.float32).at[:, :OUT_DIM].set(
                params['Wc'])
            bc = jnp.zeros((1, D), jnp.float32).at[0, :OUT_DIM].set(
                params['bc'])
            h, cls = _tc_mlp_final(parts[0], parts[1], cp['W1'], b1,
                                   cp['W2'], b2, scale, shift, wc, bc)
    return cls[:N_NODES, :OUT_DIM]
